# Initial kernel scaffold; baseline (speedup 1.0000x reference)
import jax
import jax.numpy as jnp
from jax.experimental import pallas as pl


def kernel(x, padding_mask, gate_w, fc1_w, fc1_b, fc2_w, fc2_b):
    raise NotImplementedError("write your pallas kernel here")


# speedup vs baseline: 1.6686x; 1.6686x over previous
import jax
import jax.numpy as jnp
from jax.experimental import pallas as pl
from jax.experimental.pallas import tpu as pltpu

_B, _T, _D, _H, _E = 2, 2048, 768, 768, 8
_NT = _B * _T
_TCH = 1024
_NCH = _NT // _TCH


def _moe_body(x_ref, gw_ref, m_ref, w1_ref, b1_ref, w2_ref, b2_ref,
              out_ref, gate_s):
    e = pl.program_id(1)

    @pl.when(e == 0)
    def _compute_gate():
        logits = jax.lax.dot_general(
            x_ref[...], gw_ref[...], (((1,), (1,)), ((), ())),
            preferred_element_type=jnp.float32)
        mx = jnp.max(logits, axis=-1, keepdims=True)
        p = jnp.exp(logits - mx)
        p = p / jnp.sum(p, axis=-1, keepdims=True)
        p = jnp.where(m_ref[...] > 0, 0.0, p)
        iota = jax.lax.broadcasted_iota(jnp.int32, (_TCH, _E), 1)
        i1 = jnp.argmax(p, axis=-1)
        oh1 = iota == i1[:, None]
        i2 = jnp.argmax(jnp.where(oh1, -1.0, p), axis=-1)
        oh2 = iota == i2[:, None]
        sel = oh1 | oh2
        s = jnp.sum(jnp.where(sel, p, 0.0), axis=-1, keepdims=True)
        s = jnp.where(s == 0.0, 1.0, s)
        gate_s[...] = jnp.where(sel, p, 0.0) / s

    h = jax.lax.dot_general(
        x_ref[...], w1_ref[0], (((1,), (1,)), ((), ())),
        preferred_element_type=jnp.float32)
    h = jnp.maximum(h + b1_ref[0], 0.0)
    y = jax.lax.dot_general(
        h, w2_ref[0], (((1,), (1,)), ((), ())),
        preferred_element_type=jnp.float32)
    y = y + b2_ref[0]

    iota = jax.lax.broadcasted_iota(jnp.int32, (_TCH, _E), 1)
    ge = jnp.sum(jnp.where(iota == e, gate_s[...], 0.0), axis=-1,
                 keepdims=True)
    acc = ge * y

    @pl.when(e == 0)
    def _init():
        out_ref[...] = acc

    @pl.when(e > 0)
    def _accum():
        out_ref[...] += acc


def kernel(x, padding_mask, gate_w, fc1_w, fc1_b, fc2_w, fc2_b):
    xf = x.reshape(_NT, _D)
    maskf = padding_mask.reshape(_NT, 1).astype(jnp.float32)

    out = pl.pallas_call(
        _moe_body,
        grid=(_NCH, _E),
        in_specs=[
            pl.BlockSpec((_TCH, _D), lambda t, e: (t, 0)),
            pl.BlockSpec((_E, _D), lambda t, e: (0, 0)),
            pl.BlockSpec((_TCH, 1), lambda t, e: (t, 0)),
            pl.BlockSpec((1, _H, _D), lambda t, e: (e, 0, 0)),
            pl.BlockSpec((1, 1, _H), lambda t, e: (e, 0, 0)),
            pl.BlockSpec((1, _D, _H), lambda t, e: (e, 0, 0)),
            pl.BlockSpec((1, 1, _D), lambda t, e: (e, 0, 0)),
        ],
        out_specs=pl.BlockSpec((_TCH, _D), lambda t, e: (t, 0)),
        out_shape=jax.ShapeDtypeStruct((_NT, _D), jnp.float32),
        scratch_shapes=[pltpu.VMEM((_TCH, _E), jnp.float32)],
    )(xf, gate_w, maskf, fc1_w, fc1_b.reshape(_E, 1, _H),
      fc2_w, fc2_b.reshape(_E, 1, _D))

    return out.reshape(_B, _T, _D)
